# baseline (speedup 1.0000x reference)
import functools

import jax
import jax.numpy as jnp
from jax import lax
from jax.experimental import pallas as pl
from jax.experimental.pallas import tpu as pltpu
from jax.experimental.pallas import tpu_sc as plsc

BETA = 0.25


def _sc_gather(table, idx):
    V, D = table.shape
    (B,) = idx.shape
    info = plsc.get_sparse_core_info()
    NC, NS = info.num_cores, info.num_subcores
    NW = NC * NS
    assert B % (8 * NW) == 0 and D % info.num_lanes == 0
    b_per_w = B // NW
    mesh = plsc.VectorSubcoreMesh(core_axis_name="c", subcore_axis_name="s")

    @functools.partial(
        pl.kernel,
        mesh=mesh,
        out_type=jax.ShapeDtypeStruct((B, D), table.dtype),
        scratch_types=[
            pltpu.VMEM((b_per_w,), jnp.int32),
            pltpu.VMEM((b_per_w, D), jnp.float32),
            pltpu.SemaphoreType.DMA,
        ],
        compiler_params=pltpu.CompilerParams(use_tc_tiling_on_sc=False),
    )
    def k(table_hbm, idx_hbm, out_hbm, idx_v, rows_v, sem):
        wid = lax.axis_index("s") * NC + lax.axis_index("c")
        base = wid * b_per_w
        pltpu.sync_copy(idx_hbm.at[pl.ds(base, b_per_w)], idx_v)
        pltpu.async_copy(table_hbm.at[idx_v], rows_v, sem).wait()
        pltpu.sync_copy(rows_v, out_hbm.at[pl.ds(base, b_per_w)])

    return k(table, idx)


def _bdot(a, b):
    return jnp.dot(a.astype(jnp.bfloat16), b.astype(jnp.bfloat16),
                   preferred_element_type=jnp.float32)


def _enc_body(x_ref, cf_ref, w0x_ref, w0c_ref, b0_ref, w1_ref, b1_ref,
              w2_ref, b2_ref, w3_ref, b3_ref, z_ref):
    h = (_bdot(x_ref[...], w0x_ref[...])
         + _bdot(cf_ref[...], w0c_ref[...])
         + b0_ref[...])
    h = jnp.maximum(h, 0.0)
    h = _bdot(h, w1_ref[...]) + b1_ref[...]
    h = jnp.maximum(h, 0.0)
    h = _bdot(h, w2_ref[...]) + b2_ref[...]
    h = jnp.maximum(h, 0.0)
    z_ref[...] = _bdot(h, w3_ref[...]) + b3_ref[...]


def _encoder(x, cf, enc_params, bt):
    B, in_dim = x.shape
    _, cf_dim = cf.shape
    wts = [w.T for (w, _) in enc_params]
    bs = [b[None, :] for (_, b) in enc_params]
    w0x, w0c = wts[0][:in_dim], wts[0][in_dim:]
    e_dim = wts[3].shape[1]
    grid = (B // bt,)

    def row_spec(d):
        return pl.BlockSpec((bt, d), lambda i: (i, 0))

    def full_spec(a):
        return pl.BlockSpec(a.shape, lambda i: (0,) * a.ndim)

    consts = [w0x, w0c, bs[0], wts[1], bs[1], wts[2], bs[2], wts[3], bs[3]]
    return pl.pallas_call(
        _enc_body,
        grid=grid,
        in_specs=[row_spec(in_dim), row_spec(cf_dim)] + [full_spec(a) for a in consts],
        out_specs=row_spec(e_dim),
        out_shape=jax.ShapeDtypeStruct((B, e_dim), jnp.float32),
        compiler_params=pltpu.CompilerParams(
            dimension_semantics=("arbitrary",)),
    )(x, cf, *consts)


def _rq_body(z_ref, cb_ref, cbt_ref, csq_ref, xq_ref, idx_ref, loss_ref):
    L = cb_ref.shape[0]
    n_emb = cb_ref.shape[1]
    r = z_ref[...]
    bt = r.shape[0]
    iota = lax.broadcasted_iota(jnp.int32, (bt, n_emb), 1)
    xq = jnp.zeros_like(r)
    loss_blk = jnp.zeros(loss_ref.shape, jnp.float32)
    liota = lax.broadcasted_iota(jnp.int32, loss_ref.shape, 1)
    idx_cols = []
    for l in range(L):
        m = _bdot(r, cbt_ref[l])
        rsq = jnp.sum(r * r, axis=1, keepdims=True)
        d = rsq - 2.0 * m + csq_ref[l][None, :]
        dmin = jnp.min(d, axis=1, keepdims=True)
        idx = jnp.min(jnp.where(d == dmin, iota, n_emb), axis=1)
        oh = (iota == idx[:, None]).astype(jnp.float32)
        q = jnp.dot(oh, cb_ref[l], preferred_element_type=jnp.float32,
                    precision=jax.lax.Precision.HIGHEST)
        loss_blk = loss_blk + jnp.where(
            liota == l, jnp.sum((q - r) ** 2), 0.0)
        xq = xq + q
        r = r - q
        idx_cols.append(idx[:, None])
    xq_ref[...] = xq
    idx_ref[...] = jnp.concatenate(idx_cols, axis=1)
    loss_ref[...] = loss_blk


def _rq(z, codebooks, bt):
    B, e_dim = z.shape
    L, n_emb, _ = codebooks.shape
    cbt = jnp.transpose(codebooks, (0, 2, 1))
    csq = jnp.sum(codebooks ** 2, axis=2)
    grid = (B // bt,)

    def full_spec(a):
        return pl.BlockSpec(a.shape, lambda i: (0,) * a.ndim)

    xq, idx, loss = pl.pallas_call(
        _rq_body,
        grid=grid,
        in_specs=[pl.BlockSpec((bt, e_dim), lambda i: (i, 0)),
                  full_spec(codebooks), full_spec(cbt), full_spec(csq)],
        out_specs=[pl.BlockSpec((bt, e_dim), lambda i: (i, 0)),
                   pl.BlockSpec((bt, L), lambda i: (i, 0)),
                   pl.BlockSpec((1, 8, 128), lambda i: (i, 0, 0))],
        out_shape=[jax.ShapeDtypeStruct((B, e_dim), jnp.float32),
                   jax.ShapeDtypeStruct((B, L), jnp.int32),
                   jax.ShapeDtypeStruct((B // bt, 8, 128), jnp.float32)],
        compiler_params=pltpu.CompilerParams(
            dimension_semantics=("arbitrary",)),
    )(z, codebooks, cbt, csq)
    level_sums = jnp.sum(loss, axis=0)[:L, 0]
    losses = (1.0 + BETA) * level_sums / (B * e_dim)
    rq_loss = jnp.mean(losses)
    return xq, rq_loss, idx


def _dec_body(xq_ref, w0_ref, b0_ref, w1_ref, b1_ref, w2_ref, b2_ref,
              w3_ref, b3_ref, out_ref):
    h = _bdot(xq_ref[...], w0_ref[...]) + b0_ref[...]
    h = jnp.maximum(h, 0.0)
    h = _bdot(h, w1_ref[...]) + b1_ref[...]
    h = jnp.maximum(h, 0.0)
    h = _bdot(h, w2_ref[...]) + b2_ref[...]
    h = jnp.maximum(h, 0.0)
    out_ref[...] = _bdot(h, w3_ref[...]) + b3_ref[...]


def _decoder(xq, dec_params, bt):
    B, e_dim = xq.shape
    wts = [w.T for (w, _) in dec_params]
    bs = [b[None, :] for (_, b) in dec_params]
    out_dim = wts[3].shape[1]
    grid = (B // bt,)

    def full_spec(a):
        return pl.BlockSpec(a.shape, lambda i: (0,) * a.ndim)

    consts = [wts[0], bs[0], wts[1], bs[1], wts[2], bs[2], wts[3], bs[3]]
    return pl.pallas_call(
        _dec_body,
        grid=grid,
        in_specs=[pl.BlockSpec((bt, e_dim), lambda i: (i, 0))]
        + [full_spec(a) for a in consts],
        out_specs=pl.BlockSpec((bt, out_dim), lambda i: (i, 0)),
        out_shape=jax.ShapeDtypeStruct((B, out_dim), jnp.float32),
        compiler_params=pltpu.CompilerParams(
            dimension_semantics=("arbitrary",)),
    )(xq, *consts)


def kernel(x, labels, emb_idx, cf_embedding, enc_params, dec_params, codebooks):
    del labels
    cf = _sc_gather(cf_embedding, emb_idx)
    z = _encoder(x, cf, enc_params, bt=512)
    xq, rq_loss, indices = _rq(z, codebooks, bt=512)
    out = _decoder(xq, dec_params, bt=512)
    return (out, rq_loss, indices, xq)


# speedup vs baseline: 1.0642x; 1.0642x over previous
import functools

import jax
import jax.numpy as jnp
from jax import lax
from jax.experimental import pallas as pl
from jax.experimental.pallas import tpu as pltpu
from jax.experimental.pallas import tpu_sc as plsc

BETA = 0.25


def _sc_gather(table, idx):
    V, D = table.shape
    (B,) = idx.shape
    info = plsc.get_sparse_core_info()
    NC, NS = info.num_cores, info.num_subcores
    NW = NC * NS
    assert B % (8 * NW) == 0 and D % info.num_lanes == 0
    b_per_w = B // NW
    mesh = plsc.VectorSubcoreMesh(core_axis_name="c", subcore_axis_name="s")

    @functools.partial(
        pl.kernel,
        mesh=mesh,
        out_type=jax.ShapeDtypeStruct((B, D), table.dtype),
        scratch_types=[
            pltpu.VMEM((b_per_w,), jnp.int32),
            pltpu.VMEM((b_per_w, D), jnp.float32),
            pltpu.SemaphoreType.DMA,
        ],
        compiler_params=pltpu.CompilerParams(use_tc_tiling_on_sc=False),
    )
    def k(table_hbm, idx_hbm, out_hbm, idx_v, rows_v, sem):
        wid = lax.axis_index("s") * NC + lax.axis_index("c")
        base = wid * b_per_w
        pltpu.sync_copy(idx_hbm.at[pl.ds(base, b_per_w)], idx_v)
        pltpu.async_copy(table_hbm.at[idx_v], rows_v, sem).wait()
        pltpu.sync_copy(rows_v, out_hbm.at[pl.ds(base, b_per_w)])

    return k(table, idx)


def _bdot(a, b):
    return jnp.dot(a.astype(jnp.bfloat16), b.astype(jnp.bfloat16),
                   preferred_element_type=jnp.float32)


def _bdot_t(a, w):
    return jax.lax.dot_general(
        a.astype(jnp.bfloat16), w.astype(jnp.bfloat16),
        (((1,), (1,)), ((), ())), preferred_element_type=jnp.float32)


def _enc_body(x_ref, cf_ref, w0_ref, b0_ref, w1_ref, b1_ref,
              w2_ref, b2_ref, w3_ref, b3_ref, z_ref):
    h0 = jnp.concatenate([x_ref[...], cf_ref[...]], axis=1)
    h = _bdot_t(h0, w0_ref[...]) + b0_ref[...]
    h = jnp.maximum(h, 0.0)
    h = _bdot_t(h, w1_ref[...]) + b1_ref[...]
    h = jnp.maximum(h, 0.0)
    h = _bdot_t(h, w2_ref[...]) + b2_ref[...]
    h = jnp.maximum(h, 0.0)
    z_ref[...] = _bdot_t(h, w3_ref[...]) + b3_ref[...]


def _encoder(x, cf, enc_params, bt):
    B, in_dim = x.shape
    _, cf_dim = cf.shape
    ws = [w for (w, _) in enc_params]
    bs = [b[None, :] for (_, b) in enc_params]
    e_dim = ws[3].shape[0]
    grid = (B // bt,)

    def row_spec(d):
        return pl.BlockSpec((bt, d), lambda i: (i, 0))

    def full_spec(a):
        return pl.BlockSpec(a.shape, lambda i: (0,) * a.ndim)

    consts = [ws[0], bs[0], ws[1], bs[1], ws[2], bs[2], ws[3], bs[3]]
    return pl.pallas_call(
        _enc_body,
        grid=grid,
        in_specs=[row_spec(in_dim), row_spec(cf_dim)] + [full_spec(a) for a in consts],
        out_specs=row_spec(e_dim),
        out_shape=jax.ShapeDtypeStruct((B, e_dim), jnp.float32),
        compiler_params=pltpu.CompilerParams(
            dimension_semantics=("arbitrary",)),
    )(x, cf, *consts)


def _rq_body(z_ref, cb_ref, csq_ref, xq_ref, idx_ref, loss_ref):
    L = cb_ref.shape[0]
    n_emb = cb_ref.shape[1]
    r = z_ref[...]
    bt = r.shape[0]
    iota = lax.broadcasted_iota(jnp.int32, (bt, n_emb), 1)
    xq = jnp.zeros_like(r)
    loss_blk = jnp.zeros(loss_ref.shape, jnp.float32)
    liota = lax.broadcasted_iota(jnp.int32, loss_ref.shape, 1)
    idx_cols = []
    for l in range(L):
        m = _bdot_t(r, cb_ref[l])
        rsq = jnp.sum(r * r, axis=1, keepdims=True)
        d = rsq - 2.0 * m + csq_ref[l][None, :]
        dmin = jnp.min(d, axis=1, keepdims=True)
        idx = jnp.min(jnp.where(d == dmin, iota, n_emb), axis=1)
        oh = (iota == idx[:, None]).astype(jnp.float32)
        q = jnp.dot(oh, cb_ref[l], preferred_element_type=jnp.float32,
                    precision=jax.lax.Precision.HIGHEST)
        loss_blk = loss_blk + jnp.where(
            liota == l, jnp.sum((q - r) ** 2), 0.0)
        xq = xq + q
        r = r - q
        idx_cols.append(idx[:, None])
    xq_ref[...] = xq
    idx_ref[...] = jnp.concatenate(idx_cols, axis=1)
    loss_ref[...] = loss_blk


def _rq(z, codebooks, bt):
    B, e_dim = z.shape
    L, n_emb, _ = codebooks.shape
    csq = jnp.sum(codebooks ** 2, axis=2)
    grid = (B // bt,)

    def full_spec(a):
        return pl.BlockSpec(a.shape, lambda i: (0,) * a.ndim)

    xq, idx, loss = pl.pallas_call(
        _rq_body,
        grid=grid,
        in_specs=[pl.BlockSpec((bt, e_dim), lambda i: (i, 0)),
                  full_spec(codebooks), full_spec(csq)],
        out_specs=[pl.BlockSpec((bt, e_dim), lambda i: (i, 0)),
                   pl.BlockSpec((bt, L), lambda i: (i, 0)),
                   pl.BlockSpec((1, 8, 128), lambda i: (i, 0, 0))],
        out_shape=[jax.ShapeDtypeStruct((B, e_dim), jnp.float32),
                   jax.ShapeDtypeStruct((B, L), jnp.int32),
                   jax.ShapeDtypeStruct((B // bt, 8, 128), jnp.float32)],
        compiler_params=pltpu.CompilerParams(
            dimension_semantics=("arbitrary",)),
    )(z, codebooks, csq)
    level_sums = jnp.sum(loss, axis=0)[:L, 0]
    losses = (1.0 + BETA) * level_sums / (B * e_dim)
    rq_loss = jnp.mean(losses)
    return xq, rq_loss, idx


def _dec_body(xq_ref, w0_ref, b0_ref, w1_ref, b1_ref, w2_ref, b2_ref,
              w3_ref, b3_ref, out_ref):
    h = _bdot_t(xq_ref[...], w0_ref[...]) + b0_ref[...]
    h = jnp.maximum(h, 0.0)
    h = _bdot_t(h, w1_ref[...]) + b1_ref[...]
    h = jnp.maximum(h, 0.0)
    h = _bdot_t(h, w2_ref[...]) + b2_ref[...]
    h = jnp.maximum(h, 0.0)
    out_ref[...] = _bdot_t(h, w3_ref[...]) + b3_ref[...]


def _decoder(xq, dec_params, bt):
    B, e_dim = xq.shape
    ws = [w for (w, _) in dec_params]
    bs = [b[None, :] for (_, b) in dec_params]
    out_dim = ws[3].shape[0]
    grid = (B // bt,)

    def full_spec(a):
        return pl.BlockSpec(a.shape, lambda i: (0,) * a.ndim)

    consts = [ws[0], bs[0], ws[1], bs[1], ws[2], bs[2], ws[3], bs[3]]
    return pl.pallas_call(
        _dec_body,
        grid=grid,
        in_specs=[pl.BlockSpec((bt, e_dim), lambda i: (i, 0))]
        + [full_spec(a) for a in consts],
        out_specs=pl.BlockSpec((bt, out_dim), lambda i: (i, 0)),
        out_shape=jax.ShapeDtypeStruct((B, out_dim), jnp.float32),
        compiler_params=pltpu.CompilerParams(
            dimension_semantics=("arbitrary",)),
    )(xq, *consts)


def kernel(x, labels, emb_idx, cf_embedding, enc_params, dec_params, codebooks):
    del labels
    cf = _sc_gather(cf_embedding, emb_idx)
    z = _encoder(x, cf, enc_params, bt=512)
    xq, rq_loss, indices = _rq(z, codebooks, bt=512)
    out = _decoder(xq, dec_params, bt=512)
    return (out, rq_loss, indices, xq)
